# baseline (speedup 1.0000x reference)
import functools

import jax
import jax.numpy as jnp
from jax import lax
from jax.experimental import pallas as pl
from jax.experimental.pallas import tpu as pltpu
from jax.experimental.pallas import tpu_sc as plsc

N = 10000
E = 320000
D = 128

NC = 2
NS = 16
NW = NC * NS
CH = 128
NB = 2
NCHUNK = 80
EPT = NCHUNK * CH
E_PAD = NW * EPT
NPAD = 10240
RPT = NPAD // NS
DUMMY_DST = NPAD - 8


def _seg_body(x_hbm, srcs_hbm, dsts_hbm, zeros_hbm, out_hbm, *scr):
    si = scr[0:NB]
    di = scr[NB:2 * NB]
    rows = scr[2 * NB:3 * NB]
    agg_sh = scr[3 * NB]
    isems = scr[3 * NB + 1:4 * NB + 1]
    dsems = scr[4 * NB + 1:5 * NB + 1]
    gsems = scr[5 * NB + 1:6 * NB + 1]
    ssems = scr[6 * NB + 1:7 * NB + 1]

    c = lax.axis_index("c")
    sub = lax.axis_index("s")
    tile = c * NS + sub

    pltpu.sync_copy(zeros_hbm, rows[0])
    row0 = sub * RPT
    for k in range(RPT // CH):
        pltpu.sync_copy(rows[0], agg_sh.at[pl.ds(row0 + k * CH, CH)])

    plsc.subcore_barrier()

    base = tile * EPT

    def chunk(j, carry):
        off = pl.multiple_of(base + j * CH, 8)
        pltpu.sync_copy(srcs_hbm.at[pl.ds(off, CH)], si[0])
        pltpu.sync_copy(dsts_hbm.at[pl.ds(off, CH)], di[0])
        pltpu.async_copy(x_hbm.at[si[0]], rows[0], gsems[0]).wait()
        pltpu.sync_copy(rows[0], agg_sh.at[di[0]], add=True)
        return carry

    lax.fori_loop(0, NCHUNK, chunk, 0)
    plsc.subcore_barrier()

    pltpu.sync_copy(agg_sh.at[pl.ds(row0, RPT)],
                    out_hbm.at[c, pl.ds(row0, RPT)])


_segsum = functools.partial(
    pl.kernel,
    mesh=plsc.VectorSubcoreMesh(core_axis_name="c", subcore_axis_name="s"),
    out_type=jax.ShapeDtypeStruct((NC, NPAD, D), jnp.float32),
    scratch_types=(
        [pltpu.VMEM((CH,), jnp.int32) for _ in range(2 * NB)]
        + [pltpu.VMEM((CH, D), jnp.float32) for _ in range(NB)]
        + [pltpu.VMEM_SHARED((NPAD, D), jnp.float32)]
        + [pltpu.SemaphoreType.DMA for _ in range(4 * NB)]
    ),
)(_seg_body)


BM = 1000


def _mlp_body(x_ref, p_ref, wa_ref, ba_ref, wb_ref, bb_ref, o_ref):
    t = x_ref[...] + p_ref[0] + p_ref[1]
    u = jnp.maximum(
        jnp.dot(t, wa_ref[...], preferred_element_type=jnp.float32)
        + ba_ref[...], 0.0)
    v = jnp.dot(u, wb_ref[...], preferred_element_type=jnp.float32) \
        + bb_ref[...]
    o_ref[...] = jnp.maximum(v, 0.0)


def _gin_dense(x, p, wa, ba, wb, bb):
    return pl.pallas_call(
        _mlp_body,
        grid=(N // BM,),
        in_specs=[
            pl.BlockSpec((BM, D), lambda i: (i, 0)),
            pl.BlockSpec((2, BM, D), lambda i: (0, i, 0)),
            pl.BlockSpec((D, D), lambda i: (0, 0)),
            pl.BlockSpec((1, D), lambda i: (0, 0)),
            pl.BlockSpec((D, D), lambda i: (0, 0)),
            pl.BlockSpec((1, D), lambda i: (0, 0)),
        ],
        out_specs=pl.BlockSpec((BM, D), lambda i: (i, 0)),
        out_shape=jax.ShapeDtypeStruct((N, D), jnp.float32),
    )(x, p, wa, ba, wb, bb)


@jax.jit
def kernel(x, edge_index, W0a, b0a, W0b, b0b, W1a, b1a, W1b, b1b):
    pad = E_PAD - E
    src = jnp.concatenate([edge_index[0],
                           jnp.zeros((pad,), jnp.int32)])
    dst = jnp.concatenate([edge_index[1],
                           jnp.full((pad,), DUMMY_DST, jnp.int32)])
    zeros = jnp.zeros((CH, D), jnp.float32)

    p = _segsum(x, src, dst, zeros)
    h = _gin_dense(x, p, W0a, b0a.reshape(1, D), W0b, b0b.reshape(1, D))
    p2 = _segsum(h, src, dst, zeros)
    out = _gin_dense(h, p2, W1a, b1a.reshape(1, D), W1b, b1b.reshape(1, D))
    return out


# speedup vs baseline: 1.0001x; 1.0001x over previous
import functools

import jax
import jax.numpy as jnp
from jax import lax
from jax.experimental import pallas as pl
from jax.experimental.pallas import tpu as pltpu
from jax.experimental.pallas import tpu_sc as plsc

N = 10000
E = 320000
D = 128

NC = 2
NS = 16
NW = NC * NS
CH = 128
NB = 2
NCHUNK = 80
EPT = NCHUNK * CH
E_PAD = NW * EPT
NPAD = 10240
RPT = NPAD // NS
DUMMY_DST = NPAD - 8


def _seg_body(x_hbm, srcs_hbm, dsts_hbm, zeros_hbm, out_hbm,
              src_v, dst_v, rows_v, zero_v, agg_sh, gsem):
    c = lax.axis_index("c")
    sub = lax.axis_index("s")
    tile = c * NS + sub

    pltpu.sync_copy(zeros_hbm, zero_v)
    row0 = sub * RPT
    for k in range(RPT // CH):
        pltpu.sync_copy(zero_v, agg_sh.at[pl.ds(row0 + k * CH, CH)])

    plsc.subcore_barrier()

    base = tile * EPT

    def chunk(j, carry):
        off = pl.multiple_of(base + j * CH, 8)
        pltpu.sync_copy(srcs_hbm.at[pl.ds(off, CH)], src_v)
        pltpu.sync_copy(dsts_hbm.at[pl.ds(off, CH)], dst_v)
        pltpu.async_copy(x_hbm.at[src_v], rows_v, gsem).wait()
        pltpu.sync_copy(rows_v, agg_sh.at[dst_v], add=True)
        return carry

    lax.fori_loop(0, NCHUNK, chunk, 0)
    plsc.subcore_barrier()

    pltpu.sync_copy(agg_sh.at[pl.ds(row0, RPT)],
                    out_hbm.at[c, pl.ds(row0, RPT)])


_segsum = functools.partial(
    pl.kernel,
    mesh=plsc.VectorSubcoreMesh(core_axis_name="c", subcore_axis_name="s"),
    out_type=jax.ShapeDtypeStruct((NC, NPAD, D), jnp.float32),
    scratch_types=[
        pltpu.VMEM((CH,), jnp.int32),
        pltpu.VMEM((CH,), jnp.int32),
        pltpu.VMEM((CH, D), jnp.float32),
        pltpu.VMEM((CH, D), jnp.float32),
        pltpu.VMEM_SHARED((NPAD, D), jnp.float32),
        pltpu.SemaphoreType.DMA,
    ],
)(_seg_body)


BM = 1000


def _mlp_body(x_ref, p_ref, wa_ref, ba_ref, wb_ref, bb_ref, o_ref):
    t = x_ref[...] + p_ref[0] + p_ref[1]
    u = jnp.maximum(
        jnp.dot(t, wa_ref[...], preferred_element_type=jnp.float32)
        + ba_ref[...], 0.0)
    v = jnp.dot(u, wb_ref[...], preferred_element_type=jnp.float32) \
        + bb_ref[...]
    o_ref[...] = jnp.maximum(v, 0.0)


def _gin_dense(x, p, wa, ba, wb, bb):
    return pl.pallas_call(
        _mlp_body,
        grid=(N // BM,),
        in_specs=[
            pl.BlockSpec((BM, D), lambda i: (i, 0)),
            pl.BlockSpec((2, BM, D), lambda i: (0, i, 0)),
            pl.BlockSpec((D, D), lambda i: (0, 0)),
            pl.BlockSpec((1, D), lambda i: (0, 0)),
            pl.BlockSpec((D, D), lambda i: (0, 0)),
            pl.BlockSpec((1, D), lambda i: (0, 0)),
        ],
        out_specs=pl.BlockSpec((BM, D), lambda i: (i, 0)),
        out_shape=jax.ShapeDtypeStruct((N, D), jnp.float32),
    )(x, p, wa, ba, wb, bb)


@jax.jit
def kernel(x, edge_index, W0a, b0a, W0b, b0b, W1a, b1a, W1b, b1b):
    pad = E_PAD - E
    src = jnp.concatenate([edge_index[0],
                           jnp.zeros((pad,), jnp.int32)])
    dst = jnp.concatenate([edge_index[1],
                           jnp.full((pad,), DUMMY_DST, jnp.int32)])
    zeros = jnp.zeros((CH, D), jnp.float32)

    p = _segsum(x, src, dst, zeros)
    h = _gin_dense(x, p, W0a, b0a.reshape(1, D), W0b, b0b.reshape(1, D))
    p2 = _segsum(h, src, dst, zeros)
    out = _gin_dense(h, p2, W1a, b1a.reshape(1, D), W1b, b1b.reshape(1, D))
    return out


# speedup vs baseline: 2.3067x; 2.3066x over previous
import functools

import jax
import jax.numpy as jnp
from jax import lax
from jax.experimental import pallas as pl
from jax.experimental.pallas import tpu as pltpu
from jax.experimental.pallas import tpu_sc as plsc

N = 10000
E = 320000
D = 128

NC = 2
NS = 16
NW = NC * NS
CH = 128
NB = 2
NCHUNK = 80
EPT = NCHUNK * CH
E_PAD = NW * EPT
NPAD = 10240
RPT = NPAD // NS
DUMMY_DST = NPAD - 8


def _seg_body(x_hbm, srcs_hbm, dsts_hbm, zeros_hbm, out_hbm,
              src_v, dst_v, rows_v, zero_v, agg_sh, gsem):
    c = lax.axis_index("c")
    sub = lax.axis_index("s")
    tile = c * NS + sub

    pltpu.sync_copy(zeros_hbm, zero_v)
    row0 = sub * RPT
    for k in range(RPT // CH):
        pltpu.sync_copy(zero_v, agg_sh.at[pl.ds(row0 + k * CH, CH)])

    plsc.subcore_barrier()

    base = tile * EPT

    def chunk(j, carry):
        off = pl.multiple_of(base + j * CH, 8)
        pltpu.sync_copy(srcs_hbm.at[pl.ds(off, CH)], src_v)
        pltpu.sync_copy(dsts_hbm.at[pl.ds(off, CH)], dst_v)
        pltpu.async_copy(x_hbm.at[src_v], rows_v, gsem).wait()
        pltpu.sync_copy(rows_v, agg_sh.at[dst_v], add=True)
        return carry

    lax.fori_loop(0, NCHUNK, chunk, 0)
    plsc.subcore_barrier()

    pltpu.sync_copy(agg_sh.at[pl.ds(row0, RPT)],
                    out_hbm.at[c, pl.ds(row0, RPT)])


_segsum = functools.partial(
    pl.kernel,
    mesh=plsc.VectorSubcoreMesh(core_axis_name="c", subcore_axis_name="s"),
    out_type=jax.ShapeDtypeStruct((NC, NPAD, D), jnp.float32),
    scratch_types=[
        pltpu.VMEM((CH,), jnp.int32),
        pltpu.VMEM((CH,), jnp.int32),
        pltpu.VMEM((CH, D), jnp.float32),
        pltpu.VMEM((CH, D), jnp.float32),
        pltpu.VMEM_SHARED((NPAD, D), jnp.float32),
        pltpu.SemaphoreType.DMA,
    ],
)(_seg_body)


BM = 1000


def _mlp_body(x_ref, p_ref, wa_ref, ba_ref, wb_ref, bb_ref, o_ref):
    t = x_ref[...] + p_ref[0] + p_ref[1]
    u = jnp.maximum(
        jnp.dot(t, wa_ref[...], preferred_element_type=jnp.float32)
        + ba_ref[...], 0.0)
    v = jnp.dot(u, wb_ref[...], preferred_element_type=jnp.float32) \
        + bb_ref[...]
    o_ref[...] = jnp.maximum(v, 0.0)


def _gin_dense(x, p, wa, ba, wb, bb):
    return pl.pallas_call(
        _mlp_body,
        grid=(N // BM,),
        in_specs=[
            pl.BlockSpec((BM, D), lambda i: (i, 0)),
            pl.BlockSpec((2, BM, D), lambda i: (0, i, 0)),
            pl.BlockSpec((D, D), lambda i: (0, 0)),
            pl.BlockSpec((1, D), lambda i: (0, 0)),
            pl.BlockSpec((D, D), lambda i: (0, 0)),
            pl.BlockSpec((1, D), lambda i: (0, 0)),
        ],
        out_specs=pl.BlockSpec((BM, D), lambda i: (i, 0)),
        out_shape=jax.ShapeDtypeStruct((N, D), jnp.float32),
    )(x, p, wa, ba, wb, bb)


@jax.jit
def kernel(x, edge_index, W0a, b0a, W0b, b0b, W1a, b1a, W1b, b1b):
    pad = E_PAD - E
    pad_iota = jnp.arange(pad, dtype=jnp.int32)
    src = jnp.concatenate([edge_index[0], pad_iota % N])
    dst = jnp.concatenate([edge_index[1], N + pad_iota % (NPAD - N)])
    zeros = jnp.zeros((CH, D), jnp.float32)

    p = _segsum(x, src, dst, zeros)
    h = _gin_dense(x, p, W0a, b0a.reshape(1, D), W0b, b0b.reshape(1, D))
    p2 = _segsum(h, src, dst, zeros)
    out = _gin_dense(h, p2, W1a, b1a.reshape(1, D), W1b, b1b.reshape(1, D))
    return out


# speedup vs baseline: 3.7034x; 1.6055x over previous
import functools

import jax
import jax.numpy as jnp
from jax import lax
from jax.experimental import pallas as pl
from jax.experimental.pallas import tpu as pltpu
from jax.experimental.pallas import tpu_sc as plsc

N = 10000
E = 320000
D = 128

NC = 2
NS = 16
NW = NC * NS
CH = 128
NB = 2
NCHUNK = 80
EPT = NCHUNK * CH
E_PAD = NW * EPT
NPAD = 10240
RPT = NPAD // NS
DUMMY_DST = NPAD - 8


def _seg_body(x_hbm, srcs_hbm, dsts_hbm, zeros_hbm, out_hbm, *scr):
    si = scr[0:NB]
    di = scr[NB:2 * NB]
    rows = scr[2 * NB:3 * NB]
    agg_sh = scr[3 * NB]
    isems = scr[3 * NB + 1:4 * NB + 1]
    dsems = scr[4 * NB + 1:5 * NB + 1]
    gsems = scr[5 * NB + 1:6 * NB + 1]
    ssems = scr[6 * NB + 1:7 * NB + 1]

    c = lax.axis_index("c")
    sub = lax.axis_index("s")
    tile = c * NS + sub

    pltpu.sync_copy(zeros_hbm, rows[0])
    row0 = sub * RPT
    for k in range(RPT // CH):
        pltpu.sync_copy(rows[0], agg_sh.at[pl.ds(row0 + k * CH, CH)])

    for b in range(NB):
        pltpu.async_copy(srcs_hbm.at[tile, b], si[b], isems[b])
    for b in range(NB):
        pltpu.make_async_copy(srcs_hbm.at[tile, b], si[b], isems[b]).wait()
        pltpu.async_copy(x_hbm.at[si[b]], rows[b], gsems[b])
        pltpu.async_copy(dsts_hbm.at[tile, b], di[b], dsems[b])
    plsc.subcore_barrier()

    def group(g, carry):
        j0 = g * NB
        for b in range(NB):
            j = j0 + b
            pltpu.make_async_copy(x_hbm.at[si[b]], rows[b], gsems[b]).wait()
            pltpu.make_async_copy(dsts_hbm.at[tile, j], di[b],
                                  dsems[b]).wait()
            pltpu.async_copy(rows[b], agg_sh.at[di[b]], ssems[b], add=True)

            @pl.when(j + NB < NCHUNK)
            def _():
                pltpu.async_copy(srcs_hbm.at[tile, j + NB], si[b], isems[b])
        for b in range(NB):
            j = j0 + b

            @pl.when(j + NB < NCHUNK)
            def _():
                pltpu.make_async_copy(srcs_hbm.at[tile, j + NB], si[b],
                                      isems[b]).wait()
                pltpu.make_async_copy(rows[b], agg_sh.at[di[b]],
                                      ssems[b]).wait()
                pltpu.async_copy(x_hbm.at[si[b]], rows[b], gsems[b])
                pltpu.async_copy(dsts_hbm.at[tile, j + NB], di[b], dsems[b])

            @pl.when(j + NB >= NCHUNK)
            def _():
                pltpu.make_async_copy(rows[b], agg_sh.at[di[b]],
                                      ssems[b]).wait()
        return carry

    lax.fori_loop(0, NCHUNK // NB, group, 0)
    plsc.subcore_barrier()

    pltpu.sync_copy(agg_sh.at[pl.ds(row0, RPT)],
                    out_hbm.at[c, pl.ds(row0, RPT)])


_segsum = functools.partial(
    pl.kernel,
    mesh=plsc.VectorSubcoreMesh(core_axis_name="c", subcore_axis_name="s"),
    out_type=jax.ShapeDtypeStruct((NC, NPAD, D), jnp.float32),
    scratch_types=(
        [pltpu.VMEM((CH,), jnp.int32) for _ in range(2 * NB)]
        + [pltpu.VMEM((CH, D), jnp.float32) for _ in range(NB)]
        + [pltpu.VMEM_SHARED((NPAD, D), jnp.float32)]
        + [pltpu.SemaphoreType.DMA for _ in range(4 * NB)]
    ),
)(_seg_body)


BM = 1000


def _mlp_body(x_ref, p_ref, wa_ref, ba_ref, wb_ref, bb_ref, o_ref):
    t = x_ref[...] + p_ref[0] + p_ref[1]
    u = jnp.maximum(
        jnp.dot(t, wa_ref[...], preferred_element_type=jnp.float32)
        + ba_ref[...], 0.0)
    v = jnp.dot(u, wb_ref[...], preferred_element_type=jnp.float32) \
        + bb_ref[...]
    o_ref[...] = jnp.maximum(v, 0.0)


def _gin_dense(x, p, wa, ba, wb, bb):
    return pl.pallas_call(
        _mlp_body,
        grid=(N // BM,),
        in_specs=[
            pl.BlockSpec((BM, D), lambda i: (i, 0)),
            pl.BlockSpec((2, BM, D), lambda i: (0, i, 0)),
            pl.BlockSpec((D, D), lambda i: (0, 0)),
            pl.BlockSpec((1, D), lambda i: (0, 0)),
            pl.BlockSpec((D, D), lambda i: (0, 0)),
            pl.BlockSpec((1, D), lambda i: (0, 0)),
        ],
        out_specs=pl.BlockSpec((BM, D), lambda i: (i, 0)),
        out_shape=jax.ShapeDtypeStruct((N, D), jnp.float32),
    )(x, p, wa, ba, wb, bb)


@jax.jit
def kernel(x, edge_index, W0a, b0a, W0b, b0b, W1a, b1a, W1b, b1b):
    pad = E_PAD - E
    pad_iota = jnp.arange(pad, dtype=jnp.int32)
    src = jnp.concatenate([edge_index[0], pad_iota % N]).reshape(
        NW, NCHUNK, CH)
    dst = jnp.concatenate([edge_index[1], N + pad_iota % (NPAD - N)]).reshape(
        NW, NCHUNK, CH)
    zeros = jnp.zeros((CH, D), jnp.float32)

    p = _segsum(x, src, dst, zeros)
    h = _gin_dense(x, p, W0a, b0a.reshape(1, D), W0b, b0b.reshape(1, D))
    p2 = _segsum(h, src, dst, zeros)
    out = _gin_dense(h, p2, W1a, b1a.reshape(1, D), W1b, b1b.reshape(1, D))
    return out


# speedup vs baseline: 4.2192x; 1.1393x over previous
import functools

import jax
import jax.numpy as jnp
from jax import lax
from jax.experimental import pallas as pl
from jax.experimental.pallas import tpu as pltpu
from jax.experimental.pallas import tpu_sc as plsc

N = 10000
E = 320000
D = 128

NC = 2
NS = 16
NW = NC * NS
CH = 128
NB = 3
NCHUNK = 81
EPT = NCHUNK * CH
E_PAD = NW * EPT
NPAD = 10112
RPT = NPAD // NS
DUMMY_DST = NPAD - 8


def _seg_body(x_hbm, srcs_hbm, dsts_hbm, zeros_hbm, out_hbm, *scr):
    si = scr[0:NB]
    di = scr[NB:2 * NB]
    rows = scr[2 * NB:3 * NB]
    agg_sh = scr[3 * NB]
    isems = scr[3 * NB + 1:4 * NB + 1]
    dsems = scr[4 * NB + 1:5 * NB + 1]
    gsems = scr[5 * NB + 1:6 * NB + 1]
    ssems = scr[6 * NB + 1:7 * NB + 1]

    c = lax.axis_index("c")
    sub = lax.axis_index("s")
    tile = c * NS + sub

    pltpu.sync_copy(zeros_hbm, rows[0])
    row0 = sub * RPT
    for k in range(RPT // CH):
        pltpu.sync_copy(rows[0], agg_sh.at[pl.ds(row0 + k * CH, CH)])
    rem = RPT % CH
    if rem:
        pltpu.sync_copy(rows[0].at[pl.ds(0, rem)],
                        agg_sh.at[pl.ds(row0 + (RPT // CH) * CH, rem)])

    for b in range(NB):
        pltpu.async_copy(srcs_hbm.at[tile, b], si[b], isems[b])
    for b in range(NB):
        pltpu.make_async_copy(srcs_hbm.at[tile, b], si[b], isems[b]).wait()
        pltpu.async_copy(x_hbm.at[si[b]], rows[b], gsems[b])
        pltpu.async_copy(dsts_hbm.at[tile, b], di[b], dsems[b])
    plsc.subcore_barrier()

    def group(g, carry):
        j0 = g * NB
        for b in range(NB):
            j = j0 + b
            pltpu.make_async_copy(x_hbm.at[si[b]], rows[b], gsems[b]).wait()
            pltpu.make_async_copy(dsts_hbm.at[tile, j], di[b],
                                  dsems[b]).wait()
            pltpu.async_copy(rows[b], agg_sh.at[di[b]], ssems[b], add=True)

            @pl.when(j + NB < NCHUNK)
            def _():
                pltpu.async_copy(srcs_hbm.at[tile, j + NB], si[b], isems[b])
        for b in range(NB):
            j = j0 + b

            @pl.when(j + NB < NCHUNK)
            def _():
                pltpu.make_async_copy(srcs_hbm.at[tile, j + NB], si[b],
                                      isems[b]).wait()
                pltpu.make_async_copy(rows[b], agg_sh.at[di[b]],
                                      ssems[b]).wait()
                pltpu.async_copy(x_hbm.at[si[b]], rows[b], gsems[b])
                pltpu.async_copy(dsts_hbm.at[tile, j + NB], di[b], dsems[b])

            @pl.when(j + NB >= NCHUNK)
            def _():
                pltpu.make_async_copy(rows[b], agg_sh.at[di[b]],
                                      ssems[b]).wait()
        return carry

    lax.fori_loop(0, NCHUNK // NB, group, 0)
    plsc.subcore_barrier()

    pltpu.sync_copy(agg_sh.at[pl.ds(row0, RPT)],
                    out_hbm.at[c, pl.ds(row0, RPT)])


_segsum = functools.partial(
    pl.kernel,
    mesh=plsc.VectorSubcoreMesh(core_axis_name="c", subcore_axis_name="s"),
    out_type=jax.ShapeDtypeStruct((NC, NPAD, D), jnp.float32),
    scratch_types=(
        [pltpu.VMEM((CH,), jnp.int32) for _ in range(2 * NB)]
        + [pltpu.VMEM((CH, D), jnp.float32) for _ in range(NB)]
        + [pltpu.VMEM_SHARED((NPAD, D), jnp.float32)]
        + [pltpu.SemaphoreType.DMA for _ in range(4 * NB)]
    ),
)(_seg_body)


BM = 1000


def _mlp_body(x_ref, p_ref, wa_ref, ba_ref, wb_ref, bb_ref, o_ref):
    t = x_ref[...] + p_ref[0] + p_ref[1]
    u = jnp.maximum(
        jnp.dot(t, wa_ref[...], preferred_element_type=jnp.float32)
        + ba_ref[...], 0.0)
    v = jnp.dot(u, wb_ref[...], preferred_element_type=jnp.float32) \
        + bb_ref[...]
    o_ref[...] = jnp.maximum(v, 0.0)


def _gin_dense(x, p, wa, ba, wb, bb):
    return pl.pallas_call(
        _mlp_body,
        grid=(N // BM,),
        in_specs=[
            pl.BlockSpec((BM, D), lambda i: (i, 0)),
            pl.BlockSpec((2, BM, D), lambda i: (0, i, 0)),
            pl.BlockSpec((D, D), lambda i: (0, 0)),
            pl.BlockSpec((1, D), lambda i: (0, 0)),
            pl.BlockSpec((D, D), lambda i: (0, 0)),
            pl.BlockSpec((1, D), lambda i: (0, 0)),
        ],
        out_specs=pl.BlockSpec((BM, D), lambda i: (i, 0)),
        out_shape=jax.ShapeDtypeStruct((N, D), jnp.float32),
    )(x, p, wa, ba, wb, bb)


@jax.jit
def kernel(x, edge_index, W0a, b0a, W0b, b0b, W1a, b1a, W1b, b1b):
    pad = E_PAD - E
    pad_iota = jnp.arange(pad, dtype=jnp.int32)
    src = jnp.concatenate([edge_index[0], pad_iota % N]).reshape(
        NW, NCHUNK, CH)
    dst = jnp.concatenate([edge_index[1], N + pad_iota % (NPAD - N)]).reshape(
        NW, NCHUNK, CH)
    zeros = jnp.zeros((CH, D), jnp.float32)

    p = _segsum(x, src, dst, zeros)
    h = _gin_dense(x, p, W0a, b0a.reshape(1, D), W0b, b0b.reshape(1, D))
    p2 = _segsum(h, src, dst, zeros)
    out = _gin_dense(h, p2, W1a, b1a.reshape(1, D), W1b, b1b.reshape(1, D))
    return out


# speedup vs baseline: 4.4101x; 1.0452x over previous
import functools

import jax
import jax.numpy as jnp
from jax import lax
from jax.experimental import pallas as pl
from jax.experimental.pallas import tpu as pltpu
from jax.experimental.pallas import tpu_sc as plsc

N = 10000
E = 320000
D = 128

NC = 2
NS = 16
NW = NC * NS
CH = 96
NB = 4
NCHUNK = 108
EPT = NCHUNK * CH
E_PAD = NW * EPT
NPAD = 10112
RPT = NPAD // NS
DUMMY_DST = NPAD - 8


def _seg_body(x_hbm, srcs_hbm, dsts_hbm, zeros_hbm, out_hbm, *scr):
    si = scr[0:NB]
    di = scr[NB:2 * NB]
    rows = scr[2 * NB:3 * NB]
    agg_sh = scr[3 * NB]
    isems = scr[3 * NB + 1:4 * NB + 1]
    dsems = scr[4 * NB + 1:5 * NB + 1]
    gsems = scr[5 * NB + 1:6 * NB + 1]
    ssems = scr[6 * NB + 1:7 * NB + 1]

    c = lax.axis_index("c")
    sub = lax.axis_index("s")
    tile = c * NS + sub

    pltpu.sync_copy(zeros_hbm, rows[0])
    row0 = sub * RPT
    for k in range(RPT // CH):
        pltpu.sync_copy(rows[0], agg_sh.at[pl.ds(row0 + k * CH, CH)])
    rem = RPT % CH
    if rem:
        pltpu.sync_copy(rows[0].at[pl.ds(0, rem)],
                        agg_sh.at[pl.ds(row0 + (RPT // CH) * CH, rem)])

    for b in range(NB):
        pltpu.async_copy(srcs_hbm.at[tile, b], si[b], isems[b])
    for b in range(NB):
        pltpu.make_async_copy(srcs_hbm.at[tile, b], si[b], isems[b]).wait()
        pltpu.async_copy(x_hbm.at[si[b]], rows[b], gsems[b])
        pltpu.async_copy(dsts_hbm.at[tile, b], di[b], dsems[b])
    plsc.subcore_barrier()

    def group(g, carry):
        j0 = g * NB
        for b in range(NB):
            j = j0 + b
            pltpu.make_async_copy(x_hbm.at[si[b]], rows[b], gsems[b]).wait()
            pltpu.make_async_copy(dsts_hbm.at[tile, j], di[b],
                                  dsems[b]).wait()
            pltpu.async_copy(rows[b], agg_sh.at[di[b]], ssems[b], add=True)

            @pl.when(j + NB < NCHUNK)
            def _():
                pltpu.async_copy(srcs_hbm.at[tile, j + NB], si[b], isems[b])
        for b in range(NB):
            j = j0 + b

            @pl.when(j + NB < NCHUNK)
            def _():
                pltpu.make_async_copy(srcs_hbm.at[tile, j + NB], si[b],
                                      isems[b]).wait()
                pltpu.make_async_copy(rows[b], agg_sh.at[di[b]],
                                      ssems[b]).wait()
                pltpu.async_copy(x_hbm.at[si[b]], rows[b], gsems[b])
                pltpu.async_copy(dsts_hbm.at[tile, j + NB], di[b], dsems[b])

            @pl.when(j + NB >= NCHUNK)
            def _():
                pltpu.make_async_copy(rows[b], agg_sh.at[di[b]],
                                      ssems[b]).wait()
        return carry

    lax.fori_loop(0, NCHUNK // NB, group, 0)
    plsc.subcore_barrier()

    pltpu.sync_copy(agg_sh.at[pl.ds(row0, RPT)],
                    out_hbm.at[c, pl.ds(row0, RPT)])


_segsum = functools.partial(
    pl.kernel,
    mesh=plsc.VectorSubcoreMesh(core_axis_name="c", subcore_axis_name="s"),
    out_type=jax.ShapeDtypeStruct((NC, NPAD, D), jnp.float32),
    scratch_types=(
        [pltpu.VMEM((CH,), jnp.int32) for _ in range(2 * NB)]
        + [pltpu.VMEM((CH, D), jnp.float32) for _ in range(NB)]
        + [pltpu.VMEM_SHARED((NPAD, D), jnp.float32)]
        + [pltpu.SemaphoreType.DMA for _ in range(4 * NB)]
    ),
)(_seg_body)


BM = 1000


def _mlp_body(x_ref, p_ref, wa_ref, ba_ref, wb_ref, bb_ref, o_ref):
    t = x_ref[...] + p_ref[0] + p_ref[1]
    u = jnp.maximum(
        jnp.dot(t, wa_ref[...], preferred_element_type=jnp.float32)
        + ba_ref[...], 0.0)
    v = jnp.dot(u, wb_ref[...], preferred_element_type=jnp.float32) \
        + bb_ref[...]
    o_ref[...] = jnp.maximum(v, 0.0)


def _gin_dense(x, p, wa, ba, wb, bb):
    return pl.pallas_call(
        _mlp_body,
        grid=(N // BM,),
        in_specs=[
            pl.BlockSpec((BM, D), lambda i: (i, 0)),
            pl.BlockSpec((2, BM, D), lambda i: (0, i, 0)),
            pl.BlockSpec((D, D), lambda i: (0, 0)),
            pl.BlockSpec((1, D), lambda i: (0, 0)),
            pl.BlockSpec((D, D), lambda i: (0, 0)),
            pl.BlockSpec((1, D), lambda i: (0, 0)),
        ],
        out_specs=pl.BlockSpec((BM, D), lambda i: (i, 0)),
        out_shape=jax.ShapeDtypeStruct((N, D), jnp.float32),
    )(x, p, wa, ba, wb, bb)


@jax.jit
def kernel(x, edge_index, W0a, b0a, W0b, b0b, W1a, b1a, W1b, b1b):
    pad = E_PAD - E
    pad_iota = jnp.arange(pad, dtype=jnp.int32)
    src = jnp.concatenate([edge_index[0], pad_iota % N]).reshape(
        NW, NCHUNK, CH)
    dst = jnp.concatenate([edge_index[1], N + pad_iota % (NPAD - N)]).reshape(
        NW, NCHUNK, CH)
    zeros = jnp.zeros((CH, D), jnp.float32)

    p = _segsum(x, src, dst, zeros)
    h = _gin_dense(x, p, W0a, b0a.reshape(1, D), W0b, b0b.reshape(1, D))
    p2 = _segsum(h, src, dst, zeros)
    out = _gin_dense(h, p2, W1a, b1a.reshape(1, D), W1b, b1b.reshape(1, D))
    return out


# speedup vs baseline: 4.4271x; 1.0039x over previous
import functools

import jax
import jax.numpy as jnp
from jax import lax
from jax.experimental import pallas as pl
from jax.experimental.pallas import tpu as pltpu
from jax.experimental.pallas import tpu_sc as plsc

N = 10000
E = 320000
D = 128

NC = 2
NS = 16
NW = NC * NS
CH = 72
NB = 5
NCHUNK = 145
EPT = NCHUNK * CH
E_PAD = NW * EPT
NPAD = 10112
RPT = NPAD // NS
DUMMY_DST = NPAD - 8


def _seg_body(x_hbm, srcs_hbm, dsts_hbm, zeros_hbm, out_hbm, *scr):
    si = scr[0:NB]
    di = scr[NB:2 * NB]
    rows = scr[2 * NB:3 * NB]
    agg_sh = scr[3 * NB]
    isems = scr[3 * NB + 1:4 * NB + 1]
    dsems = scr[4 * NB + 1:5 * NB + 1]
    gsems = scr[5 * NB + 1:6 * NB + 1]
    ssems = scr[6 * NB + 1:7 * NB + 1]

    c = lax.axis_index("c")
    sub = lax.axis_index("s")
    tile = c * NS + sub

    pltpu.sync_copy(zeros_hbm, rows[0])
    row0 = sub * RPT
    for k in range(RPT // CH):
        pltpu.sync_copy(rows[0], agg_sh.at[pl.ds(row0 + k * CH, CH)])
    rem = RPT % CH
    if rem:
        pltpu.sync_copy(rows[0].at[pl.ds(0, rem)],
                        agg_sh.at[pl.ds(row0 + (RPT // CH) * CH, rem)])

    for b in range(NB):
        pltpu.async_copy(srcs_hbm.at[tile, b], si[b], isems[b])
    for b in range(NB):
        pltpu.make_async_copy(srcs_hbm.at[tile, b], si[b], isems[b]).wait()
        pltpu.async_copy(x_hbm.at[si[b]], rows[b], gsems[b])
        pltpu.async_copy(dsts_hbm.at[tile, b], di[b], dsems[b])
    plsc.subcore_barrier()

    def group(g, carry):
        j0 = g * NB
        for b in range(NB):
            j = j0 + b
            pltpu.make_async_copy(x_hbm.at[si[b]], rows[b], gsems[b]).wait()
            pltpu.make_async_copy(dsts_hbm.at[tile, j], di[b],
                                  dsems[b]).wait()
            pltpu.async_copy(rows[b], agg_sh.at[di[b]], ssems[b], add=True)

            @pl.when(j + NB < NCHUNK)
            def _():
                pltpu.async_copy(srcs_hbm.at[tile, j + NB], si[b], isems[b])
        for b in range(NB):
            j = j0 + b

            @pl.when(j + NB < NCHUNK)
            def _():
                pltpu.make_async_copy(srcs_hbm.at[tile, j + NB], si[b],
                                      isems[b]).wait()
                pltpu.make_async_copy(rows[b], agg_sh.at[di[b]],
                                      ssems[b]).wait()
                pltpu.async_copy(x_hbm.at[si[b]], rows[b], gsems[b])
                pltpu.async_copy(dsts_hbm.at[tile, j + NB], di[b], dsems[b])

            @pl.when(j + NB >= NCHUNK)
            def _():
                pltpu.make_async_copy(rows[b], agg_sh.at[di[b]],
                                      ssems[b]).wait()
        return carry

    lax.fori_loop(0, NCHUNK // NB, group, 0)
    plsc.subcore_barrier()

    pltpu.sync_copy(agg_sh.at[pl.ds(row0, RPT)],
                    out_hbm.at[c, pl.ds(row0, RPT)])


_segsum = functools.partial(
    pl.kernel,
    mesh=plsc.VectorSubcoreMesh(core_axis_name="c", subcore_axis_name="s"),
    out_type=jax.ShapeDtypeStruct((NC, NPAD, D), jnp.float32),
    scratch_types=(
        [pltpu.VMEM((CH,), jnp.int32) for _ in range(2 * NB)]
        + [pltpu.VMEM((CH, D), jnp.float32) for _ in range(NB)]
        + [pltpu.VMEM_SHARED((NPAD, D), jnp.float32)]
        + [pltpu.SemaphoreType.DMA for _ in range(4 * NB)]
    ),
)(_seg_body)


BM = 1000


def _mlp_body(x_ref, p_ref, wa_ref, ba_ref, wb_ref, bb_ref, o_ref):
    t = x_ref[...] + p_ref[0] + p_ref[1]
    u = jnp.maximum(
        jnp.dot(t, wa_ref[...], preferred_element_type=jnp.float32)
        + ba_ref[...], 0.0)
    v = jnp.dot(u, wb_ref[...], preferred_element_type=jnp.float32) \
        + bb_ref[...]
    o_ref[...] = jnp.maximum(v, 0.0)


def _gin_dense(x, p, wa, ba, wb, bb):
    return pl.pallas_call(
        _mlp_body,
        grid=(N // BM,),
        in_specs=[
            pl.BlockSpec((BM, D), lambda i: (i, 0)),
            pl.BlockSpec((2, BM, D), lambda i: (0, i, 0)),
            pl.BlockSpec((D, D), lambda i: (0, 0)),
            pl.BlockSpec((1, D), lambda i: (0, 0)),
            pl.BlockSpec((D, D), lambda i: (0, 0)),
            pl.BlockSpec((1, D), lambda i: (0, 0)),
        ],
        out_specs=pl.BlockSpec((BM, D), lambda i: (i, 0)),
        out_shape=jax.ShapeDtypeStruct((N, D), jnp.float32),
    )(x, p, wa, ba, wb, bb)


@jax.jit
def kernel(x, edge_index, W0a, b0a, W0b, b0b, W1a, b1a, W1b, b1b):
    pad = E_PAD - E
    pad_iota = jnp.arange(pad, dtype=jnp.int32)
    src = jnp.concatenate([edge_index[0], pad_iota % N]).reshape(
        NW, NCHUNK, CH)
    dst = jnp.concatenate([edge_index[1], N + pad_iota % (NPAD - N)]).reshape(
        NW, NCHUNK, CH)
    zeros = jnp.zeros((CH, D), jnp.float32)

    p = _segsum(x, src, dst, zeros)
    h = _gin_dense(x, p, W0a, b0a.reshape(1, D), W0b, b0b.reshape(1, D))
    p2 = _segsum(h, src, dst, zeros)
    out = _gin_dense(h, p2, W1a, b1a.reshape(1, D), W1b, b1b.reshape(1, D))
    return out
